# baseline (device time: 33700 ns/iter reference)
import functools

import jax
import jax.numpy as jnp
from jax import lax
from jax.experimental import pallas as pl
from jax.experimental.pallas import tpu as pltpu

N_DEV = 4
B, SQ, HQ, DH = 2, 512, 8, 64
SKV = 512
BLK = 64
HALF = SKV // 2

QSCALE = 5.8 / 127.0


def kernel(x, Wq, K_ext, V_ext, Wo):
    def quant(a):
        q8 = jnp.clip(jnp.round(a * (1.0 / QSCALE)), -127.0, 127.0
                      ).astype(jnp.int8)
        return q8.reshape(B, SKV, HQ * DH)

    kq = quant(K_ext)
    vq = quant(V_ext)

    def body(x_hbm, wq_hbm, kq_ref, vq_ref, wo_hbm, out_ref,
             comm, x_vm, wq_vm, wo_vm, send_sems, recv_sems, copy_sems):
        my_i = lax.axis_index("i")
        left = (my_i - 1) % N_DEV
        right = (my_i + 1) % N_DEV

        cp_x = pltpu.make_async_copy(x_hbm, x_vm, copy_sems.at[0])
        cp_wq = pltpu.make_async_copy(wq_hbm, wq_vm, copy_sems.at[1])
        cp_wo = pltpu.make_async_copy(wo_hbm, wo_vm, copy_sems.at[2])
        cp_x.start()
        cp_wq.start()
        cp_wo.start()

        barrier_sem = pltpu.get_barrier_semaphore()
        for nbr in (left, right):
            pl.semaphore_signal(
                barrier_sem, inc=1,
                device_id=(nbr,), device_id_type=pl.DeviceIdType.MESH,
            )
        pl.semaphore_wait(barrier_sem, 2)

        chunk_a = comm.at[:, :, 0:HALF]
        chunk_b = comm.at[:, :, HALF:SKV]
        mk = functools.partial(
            pltpu.make_async_remote_copy, device_id_type=pl.DeviceIdType.MESH,
        )
        send_a1 = mk(src_ref=chunk_a, dst_ref=chunk_a, send_sem=send_sems.at[0],
                     recv_sem=recv_sems.at[0], device_id=(1,))
        send_b1 = mk(src_ref=chunk_b, dst_ref=chunk_b, send_sem=send_sems.at[1],
                     recv_sem=recv_sems.at[1], device_id=(1,))
        send_b3 = mk(src_ref=chunk_b, dst_ref=chunk_b, send_sem=send_sems.at[2],
                     recv_sem=recv_sems.at[1], device_id=(3,))
        send_a3 = mk(src_ref=chunk_a, dst_ref=chunk_a, send_sem=send_sems.at[3],
                     recv_sem=recv_sems.at[0], device_id=(3,))
        fwd_a = mk(src_ref=chunk_a, dst_ref=chunk_a, send_sem=send_sems.at[0],
                   recv_sem=recv_sems.at[0], device_id=(2,))
        fwd_b = mk(src_ref=chunk_b, dst_ref=chunk_b, send_sem=send_sems.at[1],
                   recv_sem=recv_sems.at[1], device_id=(2,))

        @pl.when(my_i == 0)
        def _():
            comm[0, :, 0:HALF] = kq_ref[:, 0:HALF]
            comm[1, :, 0:HALF] = vq_ref[:, 0:HALF]
            send_a1.start()
            send_a3.start()
            comm[0, :, HALF:SKV] = kq_ref[:, HALF:SKV]
            comm[1, :, HALF:SKV] = vq_ref[:, HALF:SKV]
            send_b3.start()
            send_b1.start()

        cp_x.wait()
        cp_wq.wait()
        wq_bf = wq_vm[...].astype(jnp.bfloat16)
        q = []
        for b in range(B):
            q_b = jnp.dot(x_vm[b].astype(jnp.bfloat16), wq_bf,
                          preferred_element_type=jnp.float32)
            q.append(q_b.astype(jnp.bfloat16))
        cp_wo.wait()
        wo_bf = wo_vm[...].astype(jnp.bfloat16)

        @pl.when(my_i == 1)
        def _():
            send_a1.wait_recv()
            fwd_a.start()

        @pl.when(my_i == 3)
        def _():
            send_b3.wait_recv()
            fwd_b.start()

        def expw(q_rows, k_bf, r0, k0, h):
            q_bh = q_rows[:, h * DH:(h + 1) * DH]
            k_bh = k_bf[:, h * DH:(h + 1) * DH]
            rows, nkv = q_bh.shape[0], k_bh.shape[0]
            scores = lax.dot_general(
                q_bh, k_bh, dimension_numbers=(((1,), (1,)), ((), ())),
                preferred_element_type=jnp.float32,
            ) * (0.125 * QSCALE)
            qb_id = (r0 + lax.broadcasted_iota(jnp.int32, (rows, nkv), 0)
                     ) // BLK
            kb_id = (k0 + lax.broadcasted_iota(jnp.int32, (rows, nkv), 1)
                     ) // BLK
            return jnp.exp(jnp.where(kb_id <= qb_id, scores, -1e9))

        @pl.when(my_i >= 2)
        def _():
            send_a1.wait_recv()
        ka = [comm[0, b, 0:HALF].astype(jnp.bfloat16) for b in range(B)]
        va = [comm[1, b, 0:HALF].astype(jnp.bfloat16) for b in range(B)]
        for b in range(B):
            heads = []
            for h in range(HQ):
                w = expw(q[b][0:HALF], ka[b], 0, 0, h)
                w = w * (QSCALE / jnp.sum(w, axis=1, keepdims=True))
                heads.append(jnp.dot(
                    w.astype(jnp.bfloat16), va[b][:, h * DH:(h + 1) * DH],
                    preferred_element_type=jnp.float32))
            ctx = jnp.concatenate(heads, axis=1)
            out_ref[b, 0:HALF] = jnp.dot(
                ctx.astype(jnp.bfloat16), wo_bf,
                preferred_element_type=jnp.float32).astype(jnp.bfloat16)

        num_a = [[None] * HQ for _ in range(B)]
        den_a = [[None] * HQ for _ in range(B)]
        for b in range(B):
            for h in range(HQ):
                w = expw(q[b][HALF:SQ], ka[b], HALF, 0, h)
                den_a[b][h] = jnp.sum(w, axis=1, keepdims=True)
                num_a[b][h] = jnp.dot(
                    w.astype(jnp.bfloat16), va[b][:, h * DH:(h + 1) * DH],
                    preferred_element_type=jnp.float32)

        @pl.when(jnp.logical_or(my_i == 1, my_i == 2))
        def _():
            send_b1.wait_recv()
        for b in range(B):
            kb = comm[0, b, HALF:SKV].astype(jnp.bfloat16)
            vb = comm[1, b, HALF:SKV].astype(jnp.bfloat16)
            heads = []
            for h in range(HQ):
                w = expw(q[b][HALF:SQ], kb, HALF, HALF, h)
                den = den_a[b][h] + jnp.sum(w, axis=1, keepdims=True)
                num = num_a[b][h] + jnp.dot(
                    w.astype(jnp.bfloat16), vb[:, h * DH:(h + 1) * DH],
                    preferred_element_type=jnp.float32)
                heads.append(num * (QSCALE / den))
            ctx = jnp.concatenate(heads, axis=1)
            out_ref[b, HALF:SQ] = jnp.dot(
                ctx.astype(jnp.bfloat16), wo_bf,
                preferred_element_type=jnp.float32).astype(jnp.bfloat16)

        @pl.when(my_i == 0)
        def _():
            send_a1.wait_send()
            send_b1.wait_send()
            send_b3.wait_send()
            send_a3.wait_send()

        @pl.when(my_i == 1)
        def _():
            fwd_a.wait_send()

        @pl.when(my_i == 3)
        def _():
            fwd_b.wait_send()

        @functools.partial(pl.run_scoped,
                           second_barrier=pltpu.SemaphoreType.REGULAR)
        def _(second_barrier):
            for nbr in (left, right):
                pl.semaphore_signal(
                    second_barrier, inc=1,
                    device_id=(nbr,), device_id_type=pl.DeviceIdType.MESH,
                )
            pl.semaphore_wait(second_barrier, 2)

    return pl.pallas_call(
        body,
        out_shape=jax.ShapeDtypeStruct((B, SQ, 768), jnp.bfloat16),
        in_specs=[
            pl.BlockSpec(memory_space=pl.ANY),
            pl.BlockSpec(memory_space=pl.ANY),
            pl.BlockSpec(memory_space=pltpu.VMEM),
            pl.BlockSpec(memory_space=pltpu.VMEM),
            pl.BlockSpec(memory_space=pl.ANY),
        ],
        out_specs=pl.BlockSpec(memory_space=pltpu.VMEM),
        scratch_shapes=[
            pltpu.VMEM((2, B, SKV, HQ * DH), jnp.int8),
            pltpu.VMEM((B, SQ, 768), jnp.float32),
            pltpu.VMEM((768, 512), jnp.float32),
            pltpu.VMEM((512, 768), jnp.float32),
            pltpu.SemaphoreType.DMA((4,)),
            pltpu.SemaphoreType.DMA((2,)),
            pltpu.SemaphoreType.DMA((3,)),
        ],
        compiler_params=pltpu.CompilerParams(collective_id=0),
    )(x, Wq, kq, vq, Wo)
